# baseline (speedup 1.0000x reference)
import functools

import jax
import jax.numpy as jnp
from jax import lax
from jax.experimental import pallas as pl
from jax.experimental.pallas import tpu as pltpu
from jax.experimental.pallas import tpu_sc as plsc

N_CLASS = 1000
BATCH = 16384
NC, NS = 2, 16
NW = NC * NS
ZWORDS = BATCH * N_CLASS
ZROWS = ZWORDS // 128
OROWS = 2 * ZROWS
ZPW = ZROWS // NW
ZCR = 80
OCR = 2 * ZCR
NCHUNK = ZPW // ZCR
UPER = 2 * N_CLASS
UEXT = UPER + 128


@functools.partial(
    pl.kernel,
    out_type=jax.ShapeDtypeStruct((OROWS, 128), jnp.float32),
    mesh=plsc.VectorSubcoreMesh(core_axis_name="c", subcore_axis_name="s"),
    scratch_types=[
        pltpu.VMEM((ZCR, 128), jnp.int32),
        pltpu.VMEM((ZCR, 128), jnp.int32),
        pltpu.VMEM((OCR, 128), jnp.float32),
        pltpu.VMEM((OCR, 128), jnp.float32),
        pltpu.VMEM((UEXT,), jnp.float32),
        pltpu.VMEM((UEXT,), jnp.float32),
        pltpu.SemaphoreType.DMA,
        pltpu.SemaphoreType.DMA,
        pltpu.SemaphoreType.DMA,
        pltpu.SemaphoreType.DMA,
    ],
)
def _zy_sc(z_hbm, u_hbm, v_hbm, out_hbm,
           z_v0, z_v1, out_v0, out_v1, u_v, v_v,
           zsem0, zsem1, osem0, osem1):
    wid = lax.axis_index("c") * NS + lax.axis_index("s")
    zrow0 = wid * ZPW
    orow0 = 2 * zrow0

    pltpu.sync_copy(u_hbm, u_v)
    pltpu.sync_copy(v_hbm, v_v)

    idx_lo = lax.iota(jnp.int32, 16) >> 1
    idx_hi = idx_lo + 8
    zbufs = (z_v0, z_v1)
    obufs = (out_v0, out_v1)
    zsems = (zsem0, zsem1)
    osems = (osem0, osem1)

    def zslice(g):
        return z_hbm.at[pl.ds(zrow0 + g * ZCR, ZCR)]

    def oslice(g):
        return out_hbm.at[pl.ds(orow0 + g * OCR, OCR)]

    pltpu.async_copy(zslice(0), z_v0, zsem0)
    pltpu.async_copy(zslice(1), z_v1, zsem1)

    def pair(i, _):
        for b in range(2):
            g = 2 * i + b
            zv, ov = zbufs[b], obufs[b]
            zs, os = zsems[b], osems[b]
            pltpu.make_async_copy(zslice(g), zv, zs).wait()

            @pl.when(i > 0)
            def _():
                pltpu.make_async_copy(ov, oslice(g - 2), os).wait()

            obflat = (orow0 + g * OCR) * 128

            @plsc.parallel_loop(0, OCR, 1)
            def row(s):
                zrow = s >> 1
                zlb = (s & 1) * 64
                uoff = (obflat + s * 128) % UPER
                for q in range(4):
                    a = zv[zrow, pl.ds(zlb + q * 16, 16)].astype(jnp.float32)
                    for h, perm in ((0, idx_lo), (1, idx_hi)):
                        vi = 2 * q + h
                        p = jnp.take_along_axis(a, perm, 0)
                        u16 = u_v[pl.ds(uoff + vi * 16, 16)]
                        v16 = v_v[pl.ds(uoff + vi * 16, 16)]
                        ov[s, pl.ds(vi * 16, 16)] = p * u16 + v16

            pltpu.async_copy(ov, oslice(g), os)

            @pl.when(g + 2 < NCHUNK)
            def _():
                pltpu.async_copy(zslice(g + 2), zv, zs)

        return 0

    lax.fori_loop(0, NCHUNK // 2, pair, 0)
    pltpu.make_async_copy(out_v0, oslice(NCHUNK - 2), osem0).wait()
    pltpu.make_async_copy(out_v1, oslice(NCHUNK - 1), osem1).wait()


def kernel(z, phi, mask):
    pf = phi.reshape(-1)
    a = mask[1, 0] - mask[0, 0] - mask[1, 1] + mask[0, 1]
    b = mask[1, 1] - mask[0, 1]
    c = mask[0, 0] - mask[0, 1]
    d = mask[0, 1]
    u = a * pf + b
    v = c * pf + d
    u_ext = jnp.concatenate([u, u[:128]])
    v_ext = jnp.concatenate([v, v[:128]])
    z2 = z.astype(jnp.int32).reshape(ZROWS, 128)
    out = _zy_sc(z2, u_ext, v_ext)
    return out.reshape(BATCH, N_CLASS, 2)


# speedup vs baseline: 89.6611x; 89.6611x over previous
import functools

import jax
import jax.numpy as jnp
from jax import lax
from jax.experimental import pallas as pl
from jax.experimental.pallas import tpu as pltpu
from jax.experimental.pallas import tpu_sc as plsc

N_CLASS = 1000
BATCH = 16384
NC, NS = 2, 16
NW = NC * NS
NBT = BATCH // 128
BTW = NBT // NW
NCT = N_CLASS // 8
ZROW = BTW * 8
OROW = 2 * BTW


@functools.partial(
    pl.kernel,
    out_type=jax.ShapeDtypeStruct((N_CLASS, 2 * NBT, 128), jnp.float32),
    mesh=plsc.VectorSubcoreMesh(core_axis_name="c", subcore_axis_name="s"),
    scratch_types=[
        pltpu.VMEM((ZROW, 128), jnp.int32),
        pltpu.VMEM((ZROW, 128), jnp.int32),
        pltpu.VMEM((8, OROW, 128), jnp.float32),
        pltpu.VMEM((8, OROW, 128), jnp.float32),
        pltpu.VMEM((2 * N_CLASS,), jnp.float32),
        pltpu.VMEM((2 * N_CLASS,), jnp.float32),
        pltpu.SemaphoreType.DMA,
        pltpu.SemaphoreType.DMA,
        pltpu.SemaphoreType.DMA,
        pltpu.SemaphoreType.DMA,
    ],
)
def _zy_sc(z_hbm, u_hbm, v_hbm, out_hbm,
           z_v0, z_v1, out_v0, out_v1, u_v, v_v,
           zsem0, zsem1, osem0, osem1):
    wid = lax.axis_index("c") * NS + lax.axis_index("s")
    bt0 = wid * BTW

    pltpu.sync_copy(u_hbm, u_v)
    pltpu.sync_copy(v_hbm, v_v)

    zbufs = (z_v0, z_v1)
    obufs = (out_v0, out_v1)
    zsems = (zsem0, zsem1)
    osems = (osem0, osem1)

    def zslice(ct):
        return z_hbm.at[ct, pl.ds(bt0 * 8, ZROW)]

    def oslice(ct):
        return out_hbm.at[pl.ds(8 * ct, 8), pl.ds(2 * bt0, OROW)]

    def compute(ct, zv, ov):
        useg = u_v[pl.ds(16 * ct, 16)]
        vseg = v_v[pl.ds(16 * ct, 16)]

        @plsc.parallel_loop(0, 8 * BTW, 1)
        def body(i):
            s = i >> 2
            bt = i & 3
            lane = jnp.full((16,), 2 * s, dtype=jnp.int32)
            bu0 = jnp.take_along_axis(useg, lane, 0)
            bv0 = jnp.take_along_axis(vseg, lane, 0)
            bu1 = jnp.take_along_axis(useg, lane + 1, 0)
            bv1 = jnp.take_along_axis(vseg, lane + 1, 0)
            for v in range(8):
                zf = zv[bt * 8 + s, pl.ds(16 * v, 16)].astype(jnp.float32)
                ov[s, 2 * bt, pl.ds(16 * v, 16)] = zf * bu0 + bv0
                ov[s, 2 * bt + 1, pl.ds(16 * v, 16)] = zf * bu1 + bv1

    pltpu.async_copy(zslice(0), z_v0, zsem0)
    pltpu.async_copy(zslice(1), z_v1, zsem1)

    def step(i, g, b):
        zv, ov = zbufs[b], obufs[b]
        zs, os = zsems[b], osems[b]
        pltpu.make_async_copy(zslice(g), zv, zs).wait()

        @pl.when(i > 0)
        def _():
            pltpu.make_async_copy(ov, oslice(g - 2), os).wait()

        compute(g, zv, ov)
        pltpu.async_copy(ov, oslice(g), os)

        @pl.when(g + 2 < NCT)
        def _():
            pltpu.async_copy(zslice(g + 2), zv, zs)

    def pair(i, _):
        step(i, 2 * i, 0)
        step(i, 2 * i + 1, 1)
        return 0

    lax.fori_loop(0, NCT // 2, pair, 0)
    step(jnp.int32(NCT // 2), jnp.int32(NCT - 1), 0)
    pltpu.make_async_copy(out_v1, oslice(NCT - 2), osem1).wait()
    pltpu.make_async_copy(out_v0, oslice(NCT - 1), osem0).wait()


def kernel(z, phi, mask):
    pf = phi.reshape(-1)
    a = mask[1, 0] - mask[0, 0] - mask[1, 1] + mask[0, 1]
    b = mask[1, 1] - mask[0, 1]
    c = mask[0, 0] - mask[0, 1]
    d = mask[0, 1]
    u = a * pf + b
    v = c * pf + d
    z3 = (
        z.T.astype(jnp.int32)
        .reshape(NCT, 8, NBT, 128)
        .transpose(0, 2, 1, 3)
        .reshape(NCT, 8 * NBT, 128)
    )
    out3 = _zy_sc(z3, u, v)
    return (
        out3.reshape(N_CLASS, NBT, 2, 128)
        .transpose(1, 3, 0, 2)
        .reshape(BATCH, N_CLASS, 2)
    )


# speedup vs baseline: 108.8631x; 1.2142x over previous
import functools

import jax
import jax.numpy as jnp
from jax import lax
from jax.experimental import pallas as pl
from jax.experimental.pallas import tpu as pltpu
from jax.experimental.pallas import tpu_sc as plsc

N_CLASS = 1000
BATCH = 16384
NC, NS = 2, 16
NW = NC * NS
NBT = BATCH // 128
BTW = NBT // NW
NCT = N_CLASS // 8
CCT = 2
NFULL = NCT // CCT
ZROW = BTW * 8
OROW = 2 * BTW


@functools.partial(
    pl.kernel,
    out_type=jax.ShapeDtypeStruct((N_CLASS, 2 * NBT, 128), jnp.float32),
    mesh=plsc.VectorSubcoreMesh(core_axis_name="c", subcore_axis_name="s"),
    scratch_types=[
        pltpu.VMEM((CCT, ZROW, 128), jnp.int32),
        pltpu.VMEM((CCT, ZROW, 128), jnp.int32),
        pltpu.VMEM((CCT * 8, OROW, 128), jnp.float32),
        pltpu.VMEM((CCT * 8, OROW, 128), jnp.float32),
        pltpu.VMEM((2 * N_CLASS,), jnp.float32),
        pltpu.VMEM((2 * N_CLASS,), jnp.float32),
        pltpu.SemaphoreType.DMA,
        pltpu.SemaphoreType.DMA,
        pltpu.SemaphoreType.DMA,
        pltpu.SemaphoreType.DMA,
    ],
)
def _zy_sc(z_hbm, u_hbm, v_hbm, out_hbm,
           z_v0, z_v1, out_v0, out_v1, u_v, v_v,
           zsem0, zsem1, osem0, osem1):
    wid = lax.axis_index("c") * NS + lax.axis_index("s")
    bt0 = wid * BTW

    pltpu.sync_copy(u_hbm, u_v)
    pltpu.sync_copy(v_hbm, v_v)

    zbufs = (z_v0, z_v1)
    obufs = (out_v0, out_v1)
    zsems = (zsem0, zsem1)
    osems = (osem0, osem1)

    def zslice(g):
        return z_hbm.at[pl.ds(CCT * g, CCT), pl.ds(bt0 * 8, ZROW)]

    def oslice(g):
        return out_hbm.at[pl.ds(8 * CCT * g, 8 * CCT), pl.ds(2 * bt0, OROW)]

    def group(ct, zv_ct, ov8):
        useg = u_v[pl.ds(16 * ct, 16)]
        vseg = v_v[pl.ds(16 * ct, 16)]

        @plsc.parallel_loop(0, 8 * BTW, 1)
        def body(i):
            s = i >> 2
            bt = i & 3
            lane = jnp.full((16,), 2 * s, dtype=jnp.int32)
            bu0 = jnp.take_along_axis(useg, lane, 0)
            bv0 = jnp.take_along_axis(vseg, lane, 0)
            bu1 = jnp.take_along_axis(useg, lane + 1, 0)
            bv1 = jnp.take_along_axis(vseg, lane + 1, 0)
            for v in range(8):
                zf = zv_ct[bt * 8 + s, pl.ds(16 * v, 16)].astype(jnp.float32)
                ov8[s, 2 * bt, pl.ds(16 * v, 16)] = zf * bu0 + bv0
                ov8[s, 2 * bt + 1, pl.ds(16 * v, 16)] = zf * bu1 + bv1

    def step(i, g, b):
        zv, ov = zbufs[b], obufs[b]
        zs, os = zsems[b], osems[b]
        pltpu.make_async_copy(zslice(g), zv, zs).wait()

        @pl.when(i > 0)
        def _():
            pltpu.make_async_copy(ov, oslice(g - 2), os).wait()

        for c in range(CCT):
            group(CCT * g + c, zv.at[c], ov.at[pl.ds(8 * c, 8)])
        pltpu.async_copy(ov, oslice(g), os)

        @pl.when(g + 2 < NFULL)
        def _():
            pltpu.async_copy(zslice(g + 2), zv, zs)

    pltpu.async_copy(zslice(0), z_v0, zsem0)
    pltpu.async_copy(zslice(1), z_v1, zsem1)

    def pair(i, _):
        step(i, 2 * i, 0)
        step(i, 2 * i + 1, 1)
        return 0

    lax.fori_loop(0, NFULL // 2, pair, 0)

    tz = z_hbm.at[pl.ds(NCT - 1, 1), pl.ds(bt0 * 8, ZROW)]
    to = out_hbm.at[pl.ds(8 * (NCT - 1), 8), pl.ds(2 * bt0, OROW)]
    pltpu.async_copy(tz, z_v0.at[pl.ds(0, 1)], zsem0)
    pltpu.make_async_copy(tz, z_v0.at[pl.ds(0, 1)], zsem0).wait()
    pltpu.make_async_copy(out_v0, oslice(NFULL - 2), osem0).wait()
    group(jnp.int32(NCT - 1), z_v0.at[0], out_v0.at[pl.ds(0, 8)])
    pltpu.async_copy(out_v0.at[pl.ds(0, 8)], to, osem0)

    pltpu.make_async_copy(out_v1, oslice(NFULL - 1), osem1).wait()
    pltpu.make_async_copy(out_v0.at[pl.ds(0, 8)], to, osem0).wait()


def kernel(z, phi, mask):
    pf = phi.reshape(-1)
    a = mask[1, 0] - mask[0, 0] - mask[1, 1] + mask[0, 1]
    b = mask[1, 1] - mask[0, 1]
    c = mask[0, 0] - mask[0, 1]
    d = mask[0, 1]
    u = a * pf + b
    v = c * pf + d
    z3 = (
        z.T.astype(jnp.int32)
        .reshape(NCT, 8, NBT, 128)
        .transpose(0, 2, 1, 3)
        .reshape(NCT, 8 * NBT, 128)
    )
    out3 = _zy_sc(z3, u, v)
    return (
        out3.reshape(N_CLASS, NBT, 2, 128)
        .transpose(1, 3, 0, 2)
        .reshape(BATCH, N_CLASS, 2)
    )
